# Initial kernel scaffold; baseline (speedup 1.0000x reference)
import jax
import jax.numpy as jnp
from jax.experimental import pallas as pl


def kernel(d_index, p_index, d_vecs, p_embeddings, y, ds_d_vecs, ds_d_edge_index, ds_d_edge_weight, ds_d_ecfps, ds_p_gos, ds_p_edge_index, ds_p_edge_weight, W1, b1, W2, b2, W3, b3, enc_W, enc_b, enc_g, enc_bt, fc1_W, fc1_b, fc_g, fc_bt, fc2_W, fc2_b):
    raise NotImplementedError("write your pallas kernel here")


# speedup vs baseline: 1.0214x; 1.0214x over previous
import jax
import jax.numpy as jnp
from jax.experimental import pallas as pl
from jax.experimental.pallas import tpu as pltpu


def _leaky(x):
    return jnp.where(x > 0, x, 0.01 * x)


def _gcn(x, ei, ew, W, b):
    N = x.shape[0]
    row = ei[0]
    col = ei[1]
    loop = jnp.arange(N)
    row_f = jnp.concatenate([row, loop])
    col_f = jnp.concatenate([col, loop])
    ew_f = jnp.concatenate([ew, jnp.full((N,), 2.0, dtype=x.dtype)])
    deg = jnp.zeros((N,), x.dtype).at[col_f].add(ew_f)
    dinv = jnp.where(deg > 0, 1.0 / jnp.sqrt(deg), 0.0)
    norm = dinv[row_f] * ew_f * dinv[col_f]
    xw = x @ W
    out = jnp.zeros((N, W.shape[1]), x.dtype).at[col_f].add(xw[row_f] * norm[:, None])
    return out + b


_BT = 512


def _row_spec(shape):
    return pl.BlockSpec((_BT, shape[1]), lambda i: (i, 0))


def _full_spec(shape):
    return pl.BlockSpec(shape, lambda i: (0,) * len(shape))


def _k1_body(dv, pemb, wt, wb, eb, raw, stats, acc):
    i = pl.program_id(0)
    t = (
        jnp.dot(dv[...], wt[...], preferred_element_type=jnp.float32)
        + jnp.dot(pemb[...], wb[...], preferred_element_type=jnp.float32)
        + eb[...]
    )
    raw[...] = t

    @pl.when(i == 0)
    def _():
        acc[...] = jnp.zeros_like(acc)

    acc[0:1, :] += jnp.sum(t, axis=0, keepdims=True)
    acc[1:2, :] += jnp.sum(t * t, axis=0, keepdims=True)

    @pl.when(i == pl.num_programs(0) - 1)
    def _():
        stats[...] = acc[...]


def _k2_body(raw, ec, go, stats, f1, f2, f3, fb, g, bt, feat, lh, stats2, acc):
    i = pl.program_id(0)
    n = raw.shape[0] * pl.num_programs(0)
    m = stats[0:1, :] / n
    var = stats[1:2, :] / n - m * m
    xn = (raw[...] - m) * jax.lax.rsqrt(var + 1e-5) * g[...] + bt[...]
    f = _leaky(xn)
    feat[...] = f
    h = (
        jnp.dot(f, f1[...], preferred_element_type=jnp.float32)
        + jnp.dot(ec[...], f2[...], preferred_element_type=jnp.float32)
        + jnp.dot(go[...], f3[...], preferred_element_type=jnp.float32)
        + fb[...]
    )
    t = _leaky(h)
    lh[...] = t

    @pl.when(i == 0)
    def _():
        acc[...] = jnp.zeros_like(acc)

    acc[0:1, :] += jnp.sum(t, axis=0, keepdims=True)
    acc[1:2, :] += jnp.sum(t * t, axis=0, keepdims=True)

    @pl.when(i == pl.num_programs(0) - 1)
    def _():
        stats2[...] = acc[...]


def _k3_body(lh, stats2, g, bt, w2, b2, yout):
    n = lh.shape[0] * pl.num_programs(0)
    m = stats2[0:1, :] / n
    var = stats2[1:2, :] / n - m * m
    xn = (lh[...] - m) * jax.lax.rsqrt(var + 1e-5) * g[...] + bt[...]
    y = jnp.dot(xn, w2[...], preferred_element_type=jnp.float32) + b2[...]
    yout[...] = jnp.maximum(y, 0.0)


def _head(dv, pemb, ec, go, enc_W, enc_b, enc_g, enc_bt, fc1_W, fc1_b,
          fc_g, fc_bt, fc2_W, fc2_b):
    B = dv.shape[0]
    nb = B // _BT
    wt, wb = enc_W[:300], enc_W[300:]
    f1, f2, f3 = fc1_W[:512], fc1_W[512:1536], fc1_W[1536:]
    eb = enc_b[None, :]
    fb = fc1_b[None, :]

    raw, stats = pl.pallas_call(
        _k1_body,
        grid=(nb,),
        in_specs=[_row_spec(dv.shape), _row_spec(pemb.shape),
                  _full_spec(wt.shape), _full_spec(wb.shape),
                  _full_spec(eb.shape)],
        out_specs=[_row_spec((B, 512)), _full_spec((2, 512))],
        out_shape=[
            jax.ShapeDtypeStruct((B, 512), jnp.float32),
            jax.ShapeDtypeStruct((2, 512), jnp.float32),
        ],
        scratch_shapes=[pltpu.VMEM((2, 512), jnp.float32)],
    )(dv, pemb, wt, wb, eb)

    feat, lh, stats2 = pl.pallas_call(
        _k2_body,
        grid=(nb,),
        in_specs=[_row_spec(raw.shape), _row_spec(ec.shape),
                  _row_spec(go.shape), _full_spec((2, 512)),
                  _full_spec(f1.shape), _full_spec(f2.shape),
                  _full_spec(f3.shape), _full_spec(fb.shape),
                  _full_spec((1, 512)), _full_spec((1, 512))],
        out_specs=[_row_spec((B, 512)), _row_spec((B, 512)),
                   _full_spec((2, 512))],
        out_shape=[
            jax.ShapeDtypeStruct((B, 512), jnp.float32),
            jax.ShapeDtypeStruct((B, 512), jnp.float32),
            jax.ShapeDtypeStruct((2, 512), jnp.float32),
        ],
        scratch_shapes=[pltpu.VMEM((2, 512), jnp.float32)],
    )(raw, ec, go, stats, f1, f2, f3, fb, enc_g[None, :], enc_bt[None, :])

    yout = pl.pallas_call(
        _k3_body,
        grid=(nb,),
        in_specs=[_row_spec(lh.shape), _full_spec((2, 512)),
                  _full_spec((1, 512)), _full_spec((1, 512)),
                  _full_spec(fc2_W.shape), _full_spec((1, 1))],
        out_specs=_row_spec((B, 1)),
        out_shape=jax.ShapeDtypeStruct((B, 1), jnp.float32),
    )(lh, stats2, fc_g[None, :], fc_bt[None, :], fc2_W, fc2_b[None, :])

    return yout, feat


def kernel(d_index, p_index, d_vecs, p_embeddings, y, ds_d_vecs,
           ds_d_edge_index, ds_d_edge_weight, ds_d_ecfps, ds_p_gos,
           ds_p_edge_index, ds_p_edge_weight, W1, b1, W2, b2, W3, b3,
           enc_W, enc_b, enc_g, enc_bt, fc1_W, fc1_b, fc_g, fc_bt,
           fc2_W, fc2_b):
    dv = _leaky(_gcn(ds_d_vecs, ds_d_edge_index, ds_d_edge_weight, W1, b1))[d_index]
    ec = _leaky(_gcn(ds_d_ecfps, ds_d_edge_index, ds_d_edge_weight, W2, b2))[d_index]
    go = _leaky(_gcn(ds_p_gos, ds_p_edge_index, ds_p_edge_weight, W3, b3))[p_index]
    return _head(dv, p_embeddings, ec, go, enc_W, enc_b, enc_g, enc_bt,
                 fc1_W, fc1_b, fc_g, fc_bt, fc2_W, fc2_b)
